# baseline (speedup 1.0000x reference)
import functools
import numpy as np
import jax
import jax.numpy as jnp
from jax import lax
from jax.experimental import pallas as pl
from jax.experimental.pallas import tpu as pltpu

N = 10000
NP = 10240
E = 160000


def _mm_att_body(x_ref, w_ref, bs_ref, bd_ref, as_ref, ad_ref, h_ref, t_ref):
    h = jnp.dot(x_ref[...], w_ref[...], preferred_element_type=jnp.float32)
    h_ref[...] = h
    hi = lax.Precision.HIGHEST
    t_ref[...] = (
        jnp.dot(h * as_ref[...], bs_ref[...], preferred_element_type=jnp.float32, precision=hi)
        + jnp.dot(h * ad_ref[...], bd_ref[...], preferred_element_type=jnp.float32, precision=hi))


def _mm_att(x, w, b_src, b_dst, a_src_flat, a_dst_flat, bm):
    m, k = x.shape
    n = w.shape[1]
    return pl.pallas_call(
        _mm_att_body,
        grid=(m // bm,),
        in_specs=[pl.BlockSpec((bm, k), lambda i: (i, 0)),
                  pl.BlockSpec((k, n), lambda i: (0, 0)),
                  pl.BlockSpec((n, 128), lambda i: (0, 0)),
                  pl.BlockSpec((n, 128), lambda i: (0, 0)),
                  pl.BlockSpec((1, n), lambda i: (0, 0)),
                  pl.BlockSpec((1, n), lambda i: (0, 0))],
        out_specs=[pl.BlockSpec((bm, n), lambda i: (i, 0)),
                   pl.BlockSpec((bm, 128), lambda i: (i, 0))],
        out_shape=[jax.ShapeDtypeStruct((m, n), jnp.float32),
                   jax.ShapeDtypeStruct((m, 128), jnp.float32)],
    )(x, w, b_src, b_dst, a_src_flat.reshape(1, n), a_dst_flat.reshape(1, n))


def _ln_relu1_body(x_ref, b_ref, w_ref, bb_ref, o_ref):
    x = x_ref[...] + b_ref[...]
    m = jnp.mean(x, axis=1, keepdims=True)
    v = jnp.mean((x - m) ** 2, axis=1, keepdims=True)
    o_ref[...] = jnp.maximum(
        (x - m) / jnp.sqrt(v + 1e-5) * w_ref[...] + bb_ref[...], 0.0)


def _ln_relu1(x, bias, w, b, bm=512):
    m, d = x.shape
    return pl.pallas_call(
        _ln_relu1_body,
        grid=(m // bm,),
        in_specs=[pl.BlockSpec((bm, d), lambda i: (i, 0)),
                  pl.BlockSpec((1, d), lambda i: (0, 0)),
                  pl.BlockSpec((1, d), lambda i: (0, 0)),
                  pl.BlockSpec((1, d), lambda i: (0, 0))],
        out_specs=pl.BlockSpec((bm, d), lambda i: (i, 0)),
        out_shape=jax.ShapeDtypeStruct((m, d), jnp.float32),
    )(x, bias.reshape(1, d), w.reshape(1, d), b.reshape(1, d))


def _ln_relu2_body(x_ref, b_ref, w_ref, bb_ref, o_ref):
    d = x_ref.shape[1]
    col = lax.broadcasted_iota(jnp.int32, (1, d), 1)
    mask = (col % 512) < 481
    x = x_ref[...] + b_ref[...]
    denom = 1443.0
    m = jnp.sum(jnp.where(mask, x, 0.0), axis=1, keepdims=True) / denom
    xc = jnp.where(mask, x - m, 0.0)
    v = jnp.sum(xc * xc, axis=1, keepdims=True) / denom
    y = xc / jnp.sqrt(v + 1e-5) * w_ref[...] + bb_ref[...]
    o_ref[...] = jnp.where(mask, jnp.maximum(y, 0.0), 0.0)


def _ln_relu2(x, bias, w, b, bm=512):
    m, d = x.shape
    return pl.pallas_call(
        _ln_relu2_body,
        grid=(m // bm,),
        in_specs=[pl.BlockSpec((bm, d), lambda i: (i, 0)),
                  pl.BlockSpec((1, d), lambda i: (0, 0)),
                  pl.BlockSpec((1, d), lambda i: (0, 0)),
                  pl.BlockSpec((1, d), lambda i: (0, 0))],
        out_specs=pl.BlockSpec((bm, d), lambda i: (i, 0)),
        out_shape=jax.ShapeDtypeStruct((m, d), jnp.float32),
    )(x, bias.reshape(1, d), w.reshape(1, d), b.reshape(1, d))


def _scale_dot_body(x_ref, c_ref, pw_ref, o_ref):
    o_ref[...] = jnp.dot(x_ref[...] * c_ref[:, :1], pw_ref[...],
                         preferred_element_type=jnp.float32)


def _scale_dot(x, c, pw, bm=2048):
    m, d = x.shape
    return pl.pallas_call(
        _scale_dot_body,
        grid=(m // bm,),
        in_specs=[pl.BlockSpec((bm, d), lambda i: (i, 0)),
                  pl.BlockSpec((bm, 128), lambda i: (i, 0)),
                  pl.BlockSpec((d, 128), lambda i: (0, 0))],
        out_specs=pl.BlockSpec((bm, 128), lambda i: (i, 0)),
        out_shape=jax.ShapeDtypeStruct((m, 128), jnp.float32),
    )(x, c, pw)


def _pool_body(w_ref, x_ref, wo_ref, bo_ref, o_ref, acc):
    i = pl.program_id(0)

    @pl.when(i == 0)
    def _():
        acc[...] = jnp.zeros_like(acc)
    acc[...] += jnp.dot(w_ref[...], x_ref[...],
                        preferred_element_type=jnp.float32,
                        precision=lax.Precision.HIGHEST)

    @pl.when(i == pl.num_programs(0) - 1)
    def _():
        o_ref[...] = jnp.dot(acc[...], wo_ref[...],
                             preferred_element_type=jnp.float32) + bo_ref[...]


def _pool(wfin, x, wo, bo_pad, bm=2048):
    m, d = x.shape
    return pl.pallas_call(
        _pool_body,
        grid=(m // bm,),
        in_specs=[pl.BlockSpec((8, bm), lambda i: (0, i)),
                  pl.BlockSpec((bm, d), lambda i: (i, 0)),
                  pl.BlockSpec((d, 512), lambda i: (0, 0)),
                  pl.BlockSpec((8, 512), lambda i: (0, 0))],
        out_specs=pl.BlockSpec((8, 512), lambda i: (0, 0)),
        out_shape=jax.ShapeDtypeStruct((8, 512), jnp.float32),
        scratch_shapes=[pltpu.VMEM((8, d), jnp.float32)],
    )(wfin, x, wo, bo_pad)


def _segment_softmax_agg(h, table, src, dst, H, C):
    a_src = table[:N, :H]
    a_dst = table[:N, 8:8 + H]
    al = a_src[src] + a_dst[dst]
    al = jnp.where(al > 0, al, 0.2 * al)
    amax = jax.ops.segment_max(al, dst, num_segments=N)
    amax = jnp.where(jnp.isfinite(amax), amax, 0.0)
    ex = jnp.exp(al - amax[dst])
    den = jax.ops.segment_sum(ex, dst, num_segments=N)
    alpha = ex / (den[dst] + 1e-16)
    hh = h[:N].reshape(N, H, C)
    out = jax.ops.segment_sum(hh[src] * alpha[:, :, None], dst, num_segments=N)
    return jnp.pad(out.reshape(N, H * C), ((0, NP - N), (0, 0)))


def _head_sel(n_cols, chunk, heads, col_off):
    b = np.zeros((n_cols, 128), np.float32)
    for g in range(heads):
        b[g * chunk:(g + 1) * chunk, col_off + g] = 1.0
    return jnp.asarray(b)


_B1_SRC = _head_sel(2048, 256, 8, 0)
_B1_DST = _head_sel(2048, 256, 8, 8)
_B2_SRC = _head_sel(1536, 512, 3, 0)
_B2_DST = _head_sel(1536, 512, 3, 8)


def kernel(token_representation, num_pos, edge_index, edge_weight,
           W1, att_src1, att_dst1, b1, bn1_w, bn1_b,
           W2, att_src2, att_dst2, b2, bn2_w, bn2_b,
           pw1, pw2, pw3, Wo, bo):
    f32 = jnp.float32
    src, dst = edge_index[0], edge_index[1]

    xp = jnp.zeros((NP, 512), f32)
    xp = xp.at[:N, :480].set(token_representation)
    xp = xp.at[:N, 480].set(num_pos[:, 0])

    W1p = jnp.zeros((512, 2048), f32).at[:481].set(W1)
    as1 = att_src1.reshape(2048)
    ad1 = att_dst1.reshape(2048)

    W2p = jnp.zeros((2048, 3, 512), f32)
    W2p = W2p.at[:, :, :481].set(W2.reshape(2048, 3, 481)).reshape(2048, 1536)
    as2 = jnp.pad(att_src2, ((0, 0), (0, 31))).reshape(1536)
    ad2 = jnp.pad(att_dst2, ((0, 0), (0, 31))).reshape(1536)

    b2p = jnp.zeros((3, 512), f32).at[:, :481].set(b2.reshape(3, 481)).reshape(1536)
    bn2w_p = jnp.zeros((3, 512), f32).at[:, :481].set(bn2_w.reshape(3, 481)).reshape(1536)
    bn2b_p = jnp.zeros((3, 512), f32).at[:, :481].set(bn2_b.reshape(3, 481)).reshape(1536)

    pws = jnp.zeros((1536, 128), f32)
    pws = pws.at[:1443, 0].set(pw1).at[:1443, 1].set(pw2).at[:1443, 2].set(pw3)
    n1, n2, n3 = (jnp.linalg.norm(pw1), jnp.linalg.norm(pw2),
                  jnp.linalg.norm(pw3))

    Woc = jnp.zeros((1536, 512), f32).at[:1443, :481].set(Wo)
    bo_pad = jnp.zeros((8, 512), f32).at[0, :481].set(bo)

    h1, t1 = _mm_att(xp, W1p, _B1_SRC, _B1_DST, as1, ad1, bm=1024)
    agg1 = _segment_softmax_agg(h1, t1, src, dst, 8, 256)
    g1 = _ln_relu1(agg1, jnp.broadcast_to(b1, (2048,)), bn1_w, bn1_b)

    h2, t2 = _mm_att(g1, W2p, _B2_SRC, _B2_DST, as2, ad2, bm=512)
    agg2 = _segment_softmax_agg(h2, t2, src, dst, 3, 512)
    g2 = _ln_relu2(agg2, b2p, bn2w_p, bn2b_p)
    g2c = jnp.concatenate(
        [g2.reshape(NP, 3, 512)[:, :, :481].reshape(NP, 1443),
         jnp.zeros((NP, 93), f32)], axis=1)

    ones = jnp.ones((NP, 128), f32)

    p1 = _scale_dot(g2c, ones, pws)[:N, 0] / n1
    s1 = jnp.tanh(p1)
    v1, perm1 = lax.top_k(s1, 5000)
    c = jnp.zeros((N,), f32).at[perm1].set(v1)
    m1 = jnp.zeros((N,), bool).at[perm1].set(True)

    cmat = jnp.broadcast_to(jnp.pad(c, (0, NP - N))[:, None], (NP, 128))
    p2 = _scale_dot(g2c, cmat, pws)[:N, 1] / n2
    s2 = jnp.where(m1, jnp.tanh(p2), -jnp.inf)
    v2, perm2 = lax.top_k(s2, 1500)
    c = c.at[perm2].multiply(v2)
    m2 = jnp.zeros((N,), bool).at[perm2].set(True)

    cmat = jnp.broadcast_to(jnp.pad(c, (0, NP - N))[:, None], (NP, 128))
    p3 = _scale_dot(g2c, cmat, pws)[:N, 2] / n3
    s3 = jnp.where(m2, jnp.tanh(p3), -jnp.inf)
    v3, perm3 = lax.top_k(s3, 300)
    c = c.at[perm3].multiply(v3)
    m3 = jnp.zeros((N,), bool).at[perm3].set(True)

    wfin = jnp.pad(jnp.where(m3, c, 0.0) / 300.0, (0, NP - N))
    wmat = jnp.zeros((8, NP), f32).at[0].set(wfin)
    outp = _pool(wmat, g2c, Woc, bo_pad)
    return outp[:1, :481]
